# baseline (speedup 1.0000x reference)
import functools

import jax
import jax.numpy as jnp
from jax import lax
from jax.experimental import pallas as pl
from jax.experimental.pallas import tpu as pltpu
from jax.experimental.pallas import tpu_sc as plsc

F32 = jnp.float32
NC, NS = 2, 16
NW = NC * NS
CH = 128
BLK = 1024
NBLK = 1000


def _silu(v):
    return v * jax.nn.sigmoid(v)


def _dot(a, b):
    return jax.lax.dot_general(a, b, (((1,), (0,)), ((), ())),
                               preferred_element_type=F32)


def _k0_body(h_ref, w1a_ref, w1b_ref, b1_ref, hr_ref, hc_ref):
    hb = h_ref[...]
    hr_ref[...] = _dot(hb, w1a_ref[...])
    hc_ref[...] = _dot(hb, w1b_ref[...]) + b1_ref[...]


def _make_gather(N, D, E_pad):
    EP = E_pad // NW
    NCHK = EP // CH
    mesh = plsc.VectorSubcoreMesh(core_axis_name="c", subcore_axis_name="s")

    NG = NCHK // 2

    @functools.partial(
        pl.kernel,
        out_type=jax.ShapeDtypeStruct((E_pad, D), F32),
        mesh=mesh,
        scratch_types=[pltpu.VMEM((CH,), jnp.int32),
                       pltpu.VMEM((CH,), jnp.int32),
                       pltpu.VMEM((CH,), jnp.int32),
                       pltpu.VMEM((CH,), jnp.int32),
                       pltpu.VMEM((CH, D), F32),
                       pltpu.VMEM((CH, D), F32),
                       pltpu.VMEM((CH, D), F32),
                       pltpu.VMEM((CH, D), F32),
                       pltpu.SemaphoreType.DMA,
                       pltpu.SemaphoreType.DMA,
                       pltpu.SemaphoreType.DMA,
                       pltpu.SemaphoreType.DMA,
                       pltpu.SemaphoreType.DMA,
                       pltpu.SemaphoreType.DMA],
    )
    def gather_k(hr, hc, rowi, coli, t1a,
                 rowv0, colv0, rowv1, colv1, bufa0, bufb0, bufa1, bufb1,
                 sI0, sI1, sG0, sG1, sO0, sO1):
        wid = lax.axis_index("s") * NC + lax.axis_index("c")
        base = wid * EP
        rv = (rowv0, rowv1)
        cv = (colv0, colv1)
        ba = (bufa0, bufa1)
        bb = (bufb0, bufb1)
        sI = (sI0, sI1)
        sG = (sG0, sG1)
        sO = (sO0, sO1)

        for p in (0, 1):
            off = base + p * CH
            pltpu.async_copy(rowi.at[pl.ds(off, CH)], rv[p], sI[p])
            pltpu.async_copy(coli.at[pl.ds(off, CH)], cv[p], sI[p])

        def outer(g, carry):
            offs = [base + (g * 2 + p) * CH for p in (0, 1)]
            for p in (0, 1):
                off = offs[p]
                pltpu.make_async_copy(
                    rowi.at[pl.ds(off, CH)], rv[p], sI[p]).wait()
                pltpu.make_async_copy(
                    coli.at[pl.ds(off, CH)], cv[p], sI[p]).wait()

                @pl.when(g > 0)
                def _():
                    pltpu.make_async_copy(
                        ba[p], t1a.at[pl.ds(off, CH)], sO[p]).wait()

                pltpu.async_copy(hr.at[rv[p]], ba[p], sG[p])
                pltpu.async_copy(hc.at[cv[p]], bb[p], sG[p])
            for p in (0, 1):
                off = offs[p]
                pltpu.make_async_copy(hr.at[rv[p]], ba[p], sG[p]).wait()
                pltpu.make_async_copy(hc.at[cv[p]], bb[p], sG[p]).wait()

                def addrow(i, carry, _p=p):
                    for j in range(D // 16):
                        sl = pl.ds(j * 16, 16)
                        ba[_p][i, sl] = ba[_p][i, sl] + bb[_p][i, sl]
                    return carry

                lax.fori_loop(0, CH, addrow, 0)
                pltpu.async_copy(ba[p], t1a.at[pl.ds(off, CH)], sO[p])

                @pl.when(g < NG - 1)
                def _():
                    offn = off + 2 * CH
                    pltpu.async_copy(rowi.at[pl.ds(offn, CH)], rv[p], sI[p])
                    pltpu.async_copy(coli.at[pl.ds(offn, CH)], cv[p], sI[p])
            return carry

        lax.fori_loop(0, NG, outer, 0)
        for p in (0, 1):
            off = base + (NCHK - 2 + p) * CH
            pltpu.make_async_copy(ba[p], t1a.at[pl.ds(off, CH)], sO[p]).wait()

    return gather_k


def _make_edge(E, D, e_base):
    def body(t1_ref, d_ref, w1c_ref, w2_ref, b2_ref,
             wa_ref, ba_ref, wc1_ref, bc1_ref, wc2_ref, msg_ref, cw_ref):
        t1 = t1_ref[...] + d_ref[...] * w1c_ref[...]
        m = _dot(_silu(t1), w2_ref[...]) + b2_ref[...]
        a = jax.nn.sigmoid(_dot(m, wa_ref[...]) + ba_ref[...])
        m = m * a
        e0 = e_base + pl.program_id(0) * BLK
        mask = (lax.broadcasted_iota(jnp.int32, (BLK, 1), 0) + e0) < E
        msg_ref[...] = jnp.where(mask, m, 0.0)
        sil = _silu(_dot(m, wc1_ref[...]) + bc1_ref[...])
        cwt = jax.lax.dot_general(wc2_ref[...], sil,
                                  (((0,), (1,)), ((), ())),
                                  preferred_element_type=F32)
        maskt = (lax.broadcasted_iota(jnp.int32, (1, BLK), 1) + e0) < E
        cw_ref[...] = jnp.where(maskt, cwt, 0.0).reshape(1, 1, BLK)

    return body


def _make_scatter(N, D, E_pad):
    EP = E_pad // NW
    NCHK = EP // CH
    RPT = (N // NS) // 8 * 8
    REM = N - RPT * NS
    mesh = plsc.VectorSubcoreMesh(core_axis_name="c", subcore_axis_name="s")

    @functools.partial(
        pl.kernel,
        out_type=(jax.ShapeDtypeStruct((N, D), F32),
                  jax.ShapeDtypeStruct((N, D), F32)),
        mesh=mesh,
        scratch_types=[pltpu.VMEM((CH,), jnp.int32),
                       pltpu.VMEM((CH,), jnp.int32),
                       pltpu.VMEM((CH, D), F32),
                       pltpu.VMEM((CH, D), F32),
                       pltpu.VMEM_SHARED((N, D), F32),
                       pltpu.SemaphoreType.DMA,
                       pltpu.SemaphoreType.DMA,
                       pltpu.SemaphoreType.DMA,
                       pltpu.SemaphoreType.DMA],
    )
    def scatter_k(vals, coli, z0, z1, p0, p1, idxv0, idxv1, vbuf0, vbuf1,
                  acc, sL0, sL1, sS0, sS1):
        c = lax.axis_index("c")
        s = lax.axis_index("s")
        wid = s * NC + c
        r0 = s * RPT
        for ci, z in ((0, z0), (1, z1)):
            @pl.when(c == ci)
            def _(z=z):
                pltpu.sync_copy(z.at[pl.ds(r0, RPT)], acc.at[pl.ds(r0, RPT)])
                if REM:
                    @pl.when(s == NS - 1)
                    def _():
                        pltpu.sync_copy(z.at[pl.ds(RPT * NS, REM)],
                                        acc.at[pl.ds(RPT * NS, REM)])
        plsc.subcore_barrier()

        base = wid * EP
        iv = (idxv0, idxv1)
        vb = (vbuf0, vbuf1)
        sL = (sL0, sL1)
        sS = (sS0, sS1)
        NG = NCHK // 2

        for p in (0, 1):
            off = base + p * CH
            pltpu.async_copy(coli.at[pl.ds(off, CH)], iv[p], sL[p])
            pltpu.async_copy(vals.at[pl.ds(off, CH)], vb[p], sL[p])

        def body(g, carry):
            offs = [base + (g * 2 + p) * CH for p in (0, 1)]
            for p in (0, 1):
                off = offs[p]
                pltpu.make_async_copy(
                    coli.at[pl.ds(off, CH)], iv[p], sL[p]).wait()
                pltpu.make_async_copy(
                    vals.at[pl.ds(off, CH)], vb[p], sL[p]).wait()
                pltpu.async_copy(vb[p], acc.at[iv[p]], sS[p], add=True)
            for p in (0, 1):
                off = offs[p]
                pltpu.make_async_copy(vb[p], acc.at[iv[p]], sS[p]).wait()

                @pl.when(g < NG - 1)
                def _():
                    offn = off + 2 * CH
                    pltpu.async_copy(coli.at[pl.ds(offn, CH)], iv[p], sL[p])
                    pltpu.async_copy(vals.at[pl.ds(offn, CH)], vb[p], sL[p])
            return carry

        lax.fori_loop(0, NG, body, 0)
        plsc.subcore_barrier()

        @pl.when(c == 0)
        def _():
            pltpu.sync_copy(acc.at[pl.ds(r0, RPT)], p0.at[pl.ds(r0, RPT)])
            if REM:
                @pl.when(s == NS - 1)
                def _():
                    pltpu.sync_copy(acc.at[pl.ds(RPT * NS, REM)],
                                    p0.at[pl.ds(RPT * NS, REM)])

        @pl.when(c == 1)
        def _():
            pltpu.sync_copy(acc.at[pl.ds(r0, RPT)], p1.at[pl.ds(r0, RPT)])
            if REM:
                @pl.when(s == NS - 1)
                def _():
                    pltpu.sync_copy(acc.at[pl.ds(RPT * NS, REM)],
                                    p1.at[pl.ds(RPT * NS, REM)])

    return scatter_k


def _make_coord_scatter(N, D, E_pad):
    EP = E_pad // NW
    NCHK = EP // CH
    RPT = (N // NS) // 8 * 8
    REM = N - RPT * NS
    mesh = plsc.VectorSubcoreMesh(core_axis_name="c", subcore_axis_name="s")

    @functools.partial(
        pl.kernel,
        out_type=(jax.ShapeDtypeStruct((N, D), F32),
                  jax.ShapeDtypeStruct((N, D), F32)),
        mesh=mesh,
        scratch_types=[pltpu.VMEM((CH,), jnp.int32),
                       pltpu.VMEM((CH,), jnp.int32),
                       pltpu.VMEM((CH,), jnp.int32),
                       pltpu.VMEM((CH,), jnp.int32),
                       pltpu.VMEM((CH,), F32),
                       pltpu.VMEM((CH,), F32),
                       pltpu.VMEM((N,), F32),
                       pltpu.VMEM((N,), F32),
                       pltpu.VMEM((N,), F32),
                       pltpu.VMEM((CH, D), F32),
                       pltpu.VMEM_SHARED((N, D), F32),
                       pltpu.SemaphoreType.DMA,
                       pltpu.SemaphoreType.DMA,
                       pltpu.SemaphoreType.DMA,
                       pltpu.SemaphoreType.DMA,
                       pltpu.SemaphoreType.DMA],
        compiler_params=pltpu.CompilerParams(needs_layout_passes=False),
    )
    def coord_k(cw1d, rowi, coli, x0, x1, x2, z0, z1, zb, p0, p1,
                rowv0, rowv1, idxv0, idxv1, cwv0, cwv1, xt0, xt1, xt2,
                cbuf, acc, sL0, sL1, sX0, sX1, sS):
        c = lax.axis_index("c")
        s = lax.axis_index("s")
        wid = s * NC + c
        r0 = s * RPT
        for ci, z in ((0, z0), (1, z1)):
            @pl.when(c == ci)
            def _(z=z):
                pltpu.sync_copy(z.at[pl.ds(r0, RPT)], acc.at[pl.ds(r0, RPT)])
                if REM:
                    @pl.when(s == NS - 1)
                    def _():
                        pltpu.sync_copy(z.at[pl.ds(RPT * NS, REM)],
                                        acc.at[pl.ds(RPT * NS, REM)])
        pltpu.sync_copy(x0, xt0)
        pltpu.sync_copy(x1, xt1)
        pltpu.sync_copy(x2, xt2)
        pltpu.sync_copy(zb.at[pl.ds(0, CH)], cbuf)
        plsc.subcore_barrier()

        base = wid * EP
        xts = (xt0, xt1, xt2)
        rv = (rowv0, rowv1)
        iv = (idxv0, idxv1)
        cw = (cwv0, cwv1)
        sL = (sL0, sL1)
        sX = (sX0, sX1)
        NG = NCHK // 2

        for p in (0, 1):
            off = base + p * CH
            pltpu.async_copy(rowi.at[pl.ds(off, CH)], rv[p], sL[p])
            pltpu.async_copy(cw1d.at[pl.ds(off, CH)], cw[p], sL[p])

        def body(g, carry):
            for p in (0, 1):
                off = base + (g * 2 + p) * CH
                pltpu.make_async_copy(
                    rowi.at[pl.ds(off, CH)], rv[p], sL[p]).wait()
                pltpu.make_async_copy(
                    cw1d.at[pl.ds(off, CH)], cw[p], sL[p]).wait()

                if p == 0:
                    @pl.when(g > 0)
                    def _():
                        pltpu.make_async_copy(cbuf, acc.at[iv[p]], sS).wait()
                else:
                    pltpu.make_async_copy(cbuf, acc.at[iv[p]], sS).wait()

                pltpu.async_copy(coli.at[pl.ds(off, CH)], iv[p], sX[p])
                for k in range(CH // 16):
                    r16 = rv[p][pl.ds(k * 16, 16)]
                    c16 = cw[p][pl.ds(k * 16, 16)]
                    e16 = lax.broadcasted_iota(jnp.int32, (16,), 0) + (k * 16)
                    for kk in range(3):
                        v16 = plsc.load_gather(xts[kk], [r16])
                        plsc.store_scatter(
                            cbuf, [e16, jnp.full((16,), kk, jnp.int32)],
                            v16 * c16)
                    plsc.store_scatter(
                        cbuf, [e16, jnp.full((16,), 3, jnp.int32)], c16)
                pltpu.make_async_copy(
                    coli.at[pl.ds(off, CH)], iv[p], sX[p]).wait()
                pltpu.async_copy(cbuf, acc.at[iv[p]], sS, add=True)

                @pl.when(g < NG - 1)
                def _():
                    offn = off + 2 * CH
                    pltpu.async_copy(rowi.at[pl.ds(offn, CH)], rv[p], sL[p])
                    pltpu.async_copy(cw1d.at[pl.ds(offn, CH)], cw[p], sL[p])
            return carry

        lax.fori_loop(0, NG, body, 0)
        pltpu.make_async_copy(cbuf, acc.at[iv[1]], sS).wait()
        plsc.subcore_barrier()

        @pl.when(c == 0)
        def _():
            pltpu.sync_copy(acc.at[pl.ds(r0, RPT)], p0.at[pl.ds(r0, RPT)])
            if REM:
                @pl.when(s == NS - 1)
                def _():
                    pltpu.sync_copy(acc.at[pl.ds(RPT * NS, REM)],
                                    p0.at[pl.ds(RPT * NS, REM)])

        @pl.when(c == 1)
        def _():
            pltpu.sync_copy(acc.at[pl.ds(r0, RPT)], p1.at[pl.ds(r0, RPT)])
            if REM:
                @pl.when(s == NS - 1)
                def _():
                    pltpu.sync_copy(acc.at[pl.ds(RPT * NS, REM)],
                                    p1.at[pl.ds(RPT * NS, REM)])

    return coord_k


def _k4_body(h_ref, pm0_ref, pm1_ref, pc0_ref, pc1_ref, x_ref,
             wn1a_ref, wn1b_ref, bn1_ref, wn2_ref, bn2_ref, g_ref, b_ref,
             hout_ref, xout_ref):
    hb = h_ref[...]
    agg = pm0_ref[...] + pm1_ref[...]
    t = _dot(_silu(_dot(hb, wn1a_ref[...]) + _dot(agg, wn1b_ref[...])
                   + bn1_ref[...]), wn2_ref[...]) + bn2_ref[...]
    pre = hb + t
    mean = jnp.mean(pre, axis=-1, keepdims=True)
    cen = pre - mean
    var = jnp.mean(cen * cen, axis=-1, keepdims=True)
    hout_ref[...] = cen * lax.rsqrt(var + 1e-5) * g_ref[...] + b_ref[...]
    aggc = pc0_ref[...] + pc1_ref[...]
    xb = x_ref[...]
    xout_ref[...] = xb + aggc[:, :3] - xb * aggc[:, 3:4]


def kernel(h, x, edge_index, edge_dist, W_e1, b_e1, W_e2, b_e2, W_a, b_a,
           W_n1, b_n1, W_n2, b_n2, W_c1, b_c1, W_c2, gamma, beta):
    N, D = h.shape
    E = edge_dist.shape[0]
    E_pad = -(-E // (NW * CH * 2)) * (NW * CH * 2)
    pad = E_pad - E

    ei = edge_index.astype(jnp.int32)
    spread = jnp.arange(pad, dtype=jnp.int32) % N
    rowp = jnp.concatenate([ei[0], spread])
    colp = jnp.concatenate([ei[1], spread])
    distp = jnp.concatenate([edge_dist, jnp.zeros((pad,), F32)])

    w1a = W_e1[:D]
    w1b = W_e1[D:2 * D]
    w1c = W_e1[2 * D:2 * D + 1]
    b1 = b_e1[None, :]

    nb = N // NBLK
    hr, hc = pl.pallas_call(
        _k0_body,
        grid=(nb,),
        in_specs=[pl.BlockSpec((NBLK, D), lambda i: (i, 0)),
                  pl.BlockSpec((D, D), lambda i: (0, 0)),
                  pl.BlockSpec((D, D), lambda i: (0, 0)),
                  pl.BlockSpec((1, D), lambda i: (0, 0))],
        out_specs=[pl.BlockSpec((NBLK, D), lambda i: (i, 0)),
                   pl.BlockSpec((NBLK, D), lambda i: (i, 0))],
        out_shape=[jax.ShapeDtypeStruct((N, D), F32),
                   jax.ShapeDtypeStruct((N, D), F32)],
    )(h, w1a, w1b, b1)

    Q = 8
    H = E_pad // Q
    eb = H // BLK
    zm = jnp.zeros((N, D), F32)
    gather_f = _make_gather(N, D, H)
    scat_f = _make_scatter(N, D, H)
    coord_f = _make_coord_scatter(N, D, H)
    pms, pcs = (zm, zm), (zm, zm)
    for hv in range(Q):
        rowh = rowp[hv * H:(hv + 1) * H]
        colh = colp[hv * H:(hv + 1) * H]
        disth = distp[hv * H:(hv + 1) * H]
        t1 = gather_f(hr, hc, rowh, colh)
        msg, cwo = pl.pallas_call(
            _make_edge(E, D, hv * H),
            grid=(eb,),
            in_specs=[pl.BlockSpec((BLK, D), lambda i: (i, 0)),
                      pl.BlockSpec((BLK, 1), lambda i: (i, 0)),
                      pl.BlockSpec((1, D), lambda i: (0, 0)),
                      pl.BlockSpec((D, D), lambda i: (0, 0)),
                      pl.BlockSpec((1, D), lambda i: (0, 0)),
                      pl.BlockSpec((D, 1), lambda i: (0, 0)),
                      pl.BlockSpec((1, 1), lambda i: (0, 0)),
                      pl.BlockSpec((D, D), lambda i: (0, 0)),
                      pl.BlockSpec((1, D), lambda i: (0, 0)),
                      pl.BlockSpec((D, 1), lambda i: (0, 0))],
            out_specs=[pl.BlockSpec((BLK, D), lambda i: (i, 0)),
                       pl.BlockSpec((1, 1, BLK), lambda i: (i, 0, 0))],
            out_shape=[jax.ShapeDtypeStruct((H, D), F32),
                       jax.ShapeDtypeStruct((eb, 1, BLK), F32)],
        )(t1, disth[:, None], w1c, W_e2, b_e2[None, :], W_a, b_a[None, :],
          W_c1, b_c1[None, :], W_c2)
        pms = scat_f(msg, colh, pms[0], pms[1])
        pcs = coord_f(cwo.reshape(H), rowh, colh,
                      x[:, 0], x[:, 1], x[:, 2], pcs[0], pcs[1], zm)
    pm0, pm1 = pms
    pc0, pc1 = pcs

    h_out, x_new = pl.pallas_call(
        _k4_body,
        grid=(nb,),
        in_specs=[pl.BlockSpec((NBLK, D), lambda i: (i, 0)),
                  pl.BlockSpec((NBLK, D), lambda i: (i, 0)),
                  pl.BlockSpec((NBLK, D), lambda i: (i, 0)),
                  pl.BlockSpec((NBLK, 128), lambda i: (i, 0)),
                  pl.BlockSpec((NBLK, 128), lambda i: (i, 0)),
                  pl.BlockSpec((NBLK, 3), lambda i: (i, 0)),
                  pl.BlockSpec((D, D), lambda i: (0, 0)),
                  pl.BlockSpec((D, D), lambda i: (0, 0)),
                  pl.BlockSpec((1, D), lambda i: (0, 0)),
                  pl.BlockSpec((D, D), lambda i: (0, 0)),
                  pl.BlockSpec((1, D), lambda i: (0, 0)),
                  pl.BlockSpec((1, D), lambda i: (0, 0)),
                  pl.BlockSpec((1, D), lambda i: (0, 0))],
        out_specs=[pl.BlockSpec((NBLK, D), lambda i: (i, 0)),
                   pl.BlockSpec((NBLK, 3), lambda i: (i, 0))],
        out_shape=[jax.ShapeDtypeStruct((N, D), F32),
                   jax.ShapeDtypeStruct((N, 3), F32)],
    )(h, pm0, pm1, pc0, pc1, x,
      W_n1[:D], W_n1[D:], b_n1[None, :],
      W_n2, b_n2[None, :], gamma[None, :], beta[None, :])

    return (h_out, x_new)


# speedup vs baseline: 1.2322x; 1.2322x over previous
import functools

import jax
import jax.numpy as jnp
from jax import lax
from jax.experimental import pallas as pl
from jax.experimental.pallas import tpu as pltpu
from jax.experimental.pallas import tpu_sc as plsc

F32 = jnp.float32
NC, NS = 2, 16
NW = NC * NS
CH = 128
BLK = 1024
NBLK = 1000


def _silu(v):
    return v * jax.nn.sigmoid(v)


def _dot(a, b):
    return jax.lax.dot_general(a, b, (((1,), (0,)), ((), ())),
                               preferred_element_type=F32)


def _k0_body(h_ref, w1a_ref, w1b_ref, b1_ref, hr_ref, hc_ref):
    hb = h_ref[...]
    hr_ref[...] = _dot(hb, w1a_ref[...])
    hc_ref[...] = _dot(hb, w1b_ref[...]) + b1_ref[...]


def _make_gather(N, D, E_pad):
    EP = E_pad // NW
    NCHK = EP // CH
    mesh = plsc.VectorSubcoreMesh(core_axis_name="c", subcore_axis_name="s")

    NG = NCHK // 2

    @functools.partial(
        pl.kernel,
        out_type=jax.ShapeDtypeStruct((E_pad, D), F32),
        mesh=mesh,
        scratch_types=[pltpu.VMEM((CH,), jnp.int32),
                       pltpu.VMEM((CH,), jnp.int32),
                       pltpu.VMEM((CH,), jnp.int32),
                       pltpu.VMEM((CH,), jnp.int32),
                       pltpu.VMEM((CH, D), F32),
                       pltpu.VMEM((CH, D), F32),
                       pltpu.VMEM((CH, D), F32),
                       pltpu.VMEM((CH, D), F32),
                       pltpu.SemaphoreType.DMA,
                       pltpu.SemaphoreType.DMA,
                       pltpu.SemaphoreType.DMA,
                       pltpu.SemaphoreType.DMA,
                       pltpu.SemaphoreType.DMA,
                       pltpu.SemaphoreType.DMA],
    )
    def gather_k(hr, hc, rowi, coli, t1a,
                 rowv0, colv0, rowv1, colv1, bufa0, bufb0, bufa1, bufb1,
                 sI0, sI1, sG0, sG1, sO0, sO1):
        wid = lax.axis_index("s") * NC + lax.axis_index("c")
        base = wid * EP
        rv = (rowv0, rowv1)
        cv = (colv0, colv1)
        ba = (bufa0, bufa1)
        bb = (bufb0, bufb1)
        sI = (sI0, sI1)
        sG = (sG0, sG1)
        sO = (sO0, sO1)

        for p in (0, 1):
            off = base + p * CH
            pltpu.async_copy(rowi.at[pl.ds(off, CH)], rv[p], sI[p])
            pltpu.async_copy(coli.at[pl.ds(off, CH)], cv[p], sI[p])

        def outer(g, carry):
            offs = [base + (g * 2 + p) * CH for p in (0, 1)]
            for p in (0, 1):
                off = offs[p]
                pltpu.make_async_copy(
                    rowi.at[pl.ds(off, CH)], rv[p], sI[p]).wait()
                pltpu.make_async_copy(
                    coli.at[pl.ds(off, CH)], cv[p], sI[p]).wait()

                @pl.when(g > 0)
                def _():
                    pltpu.make_async_copy(
                        ba[p], t1a.at[pl.ds(off, CH)], sO[p]).wait()

                pltpu.async_copy(hr.at[rv[p]], ba[p], sG[p])
                pltpu.async_copy(hc.at[cv[p]], bb[p], sG[p])
            for p in (0, 1):
                off = offs[p]
                pltpu.make_async_copy(hr.at[rv[p]], ba[p], sG[p]).wait()
                pltpu.make_async_copy(hc.at[cv[p]], bb[p], sG[p]).wait()

                def addrow(i, carry, _p=p):
                    for j in range(D // 16):
                        sl = pl.ds(j * 16, 16)
                        ba[_p][i, sl] = ba[_p][i, sl] + bb[_p][i, sl]
                    return carry

                lax.fori_loop(0, CH, addrow, 0)
                pltpu.async_copy(ba[p], t1a.at[pl.ds(off, CH)], sO[p])

                @pl.when(g < NG - 1)
                def _():
                    offn = off + 2 * CH
                    pltpu.async_copy(rowi.at[pl.ds(offn, CH)], rv[p], sI[p])
                    pltpu.async_copy(coli.at[pl.ds(offn, CH)], cv[p], sI[p])
            return carry

        lax.fori_loop(0, NG, outer, 0)
        for p in (0, 1):
            off = base + (NCHK - 2 + p) * CH
            pltpu.make_async_copy(ba[p], t1a.at[pl.ds(off, CH)], sO[p]).wait()

    return gather_k


def _make_edge(E, D, e_base):
    def body(t1_ref, d_ref, w1c_ref, w2_ref, b2_ref,
             wa_ref, ba_ref, wc1_ref, bc1_ref, wc2_ref, msg_ref, cw_ref):
        t1 = t1_ref[...] + d_ref[...] * w1c_ref[...]
        m = _dot(_silu(t1), w2_ref[...]) + b2_ref[...]
        a = jax.nn.sigmoid(_dot(m, wa_ref[...]) + ba_ref[...])
        m = m * a
        e0 = e_base + pl.program_id(0) * BLK
        mask = (lax.broadcasted_iota(jnp.int32, (BLK, 1), 0) + e0) < E
        msg_ref[...] = jnp.where(mask, m, 0.0)
        sil = _silu(_dot(m, wc1_ref[...]) + bc1_ref[...])
        cwt = jax.lax.dot_general(wc2_ref[...], sil,
                                  (((0,), (1,)), ((), ())),
                                  preferred_element_type=F32)
        maskt = (lax.broadcasted_iota(jnp.int32, (1, BLK), 1) + e0) < E
        cw_ref[...] = jnp.where(maskt, cwt, 0.0).reshape(1, 1, BLK)

    return body


def _make_scatter(N, D, E_pad):
    EP = E_pad // NW
    NCHK = EP // CH
    RPT = (N // NS) // 8 * 8
    REM = N - RPT * NS
    mesh = plsc.VectorSubcoreMesh(core_axis_name="c", subcore_axis_name="s")

    @functools.partial(
        pl.kernel,
        out_type=(jax.ShapeDtypeStruct((N, D), F32),
                  jax.ShapeDtypeStruct((N, D), F32)),
        mesh=mesh,
        scratch_types=[pltpu.VMEM((CH,), jnp.int32),
                       pltpu.VMEM((CH,), jnp.int32),
                       pltpu.VMEM((CH, D), F32),
                       pltpu.VMEM((CH, D), F32),
                       pltpu.VMEM_SHARED((N, D), F32),
                       pltpu.SemaphoreType.DMA,
                       pltpu.SemaphoreType.DMA,
                       pltpu.SemaphoreType.DMA,
                       pltpu.SemaphoreType.DMA],
    )
    def scatter_k(vals, coli, z0, z1, p0, p1, idxv0, idxv1, vbuf0, vbuf1,
                  acc, sL0, sL1, sS0, sS1):
        c = lax.axis_index("c")
        s = lax.axis_index("s")
        wid = s * NC + c
        r0 = s * RPT
        for ci, z in ((0, z0), (1, z1)):
            @pl.when(c == ci)
            def _(z=z):
                pltpu.sync_copy(z.at[pl.ds(r0, RPT)], acc.at[pl.ds(r0, RPT)])
                if REM:
                    @pl.when(s == NS - 1)
                    def _():
                        pltpu.sync_copy(z.at[pl.ds(RPT * NS, REM)],
                                        acc.at[pl.ds(RPT * NS, REM)])
        plsc.subcore_barrier()

        base = wid * EP
        iv = (idxv0, idxv1)
        vb = (vbuf0, vbuf1)
        sL = (sL0, sL1)
        sS = (sS0, sS1)
        NG = NCHK // 2

        for p in (0, 1):
            off = base + p * CH
            pltpu.async_copy(coli.at[pl.ds(off, CH)], iv[p], sL[p])
            pltpu.async_copy(vals.at[pl.ds(off, CH)], vb[p], sL[p])

        def body(g, carry):
            offs = [base + (g * 2 + p) * CH for p in (0, 1)]
            for p in (0, 1):
                off = offs[p]
                pltpu.make_async_copy(
                    coli.at[pl.ds(off, CH)], iv[p], sL[p]).wait()
                pltpu.make_async_copy(
                    vals.at[pl.ds(off, CH)], vb[p], sL[p]).wait()
                pltpu.async_copy(vb[p], acc.at[iv[p]], sS[p], add=True)
            for p in (0, 1):
                off = offs[p]
                pltpu.make_async_copy(vb[p], acc.at[iv[p]], sS[p]).wait()

                @pl.when(g < NG - 1)
                def _():
                    offn = off + 2 * CH
                    pltpu.async_copy(coli.at[pl.ds(offn, CH)], iv[p], sL[p])
                    pltpu.async_copy(vals.at[pl.ds(offn, CH)], vb[p], sL[p])
            return carry

        lax.fori_loop(0, NG, body, 0)
        plsc.subcore_barrier()

        @pl.when(c == 0)
        def _():
            pltpu.sync_copy(acc.at[pl.ds(r0, RPT)], p0.at[pl.ds(r0, RPT)])
            if REM:
                @pl.when(s == NS - 1)
                def _():
                    pltpu.sync_copy(acc.at[pl.ds(RPT * NS, REM)],
                                    p0.at[pl.ds(RPT * NS, REM)])

        @pl.when(c == 1)
        def _():
            pltpu.sync_copy(acc.at[pl.ds(r0, RPT)], p1.at[pl.ds(r0, RPT)])
            if REM:
                @pl.when(s == NS - 1)
                def _():
                    pltpu.sync_copy(acc.at[pl.ds(RPT * NS, REM)],
                                    p1.at[pl.ds(RPT * NS, REM)])

    return scatter_k


def _make_coord_scatter(N, D, E_pad):
    EP = E_pad // NW
    NCHK = EP // CH
    RPT = (N // NS) // 8 * 8
    REM = N - RPT * NS
    mesh = plsc.VectorSubcoreMesh(core_axis_name="c", subcore_axis_name="s")

    @functools.partial(
        pl.kernel,
        out_type=(jax.ShapeDtypeStruct((N, D), F32),
                  jax.ShapeDtypeStruct((N, D), F32)),
        mesh=mesh,
        scratch_types=[pltpu.VMEM((CH,), jnp.int32),
                       pltpu.VMEM((CH,), jnp.int32),
                       pltpu.VMEM((CH,), jnp.int32),
                       pltpu.VMEM((CH,), jnp.int32),
                       pltpu.VMEM((CH,), F32),
                       pltpu.VMEM((CH,), F32),
                       pltpu.VMEM((N,), F32),
                       pltpu.VMEM((N,), F32),
                       pltpu.VMEM((N,), F32),
                       pltpu.VMEM((CH, D), F32),
                       pltpu.VMEM_SHARED((N, D), F32),
                       pltpu.SemaphoreType.DMA,
                       pltpu.SemaphoreType.DMA,
                       pltpu.SemaphoreType.DMA,
                       pltpu.SemaphoreType.DMA,
                       pltpu.SemaphoreType.DMA],
        compiler_params=pltpu.CompilerParams(needs_layout_passes=False),
    )
    def coord_k(cw1d, rowi, coli, x0, x1, x2, z0, z1, zb, p0, p1,
                rowv0, rowv1, idxv0, idxv1, cwv0, cwv1, xt0, xt1, xt2,
                cbuf, acc, sL0, sL1, sX0, sX1, sS):
        c = lax.axis_index("c")
        s = lax.axis_index("s")
        wid = s * NC + c
        r0 = s * RPT
        for ci, z in ((0, z0), (1, z1)):
            @pl.when(c == ci)
            def _(z=z):
                pltpu.sync_copy(z.at[pl.ds(r0, RPT)], acc.at[pl.ds(r0, RPT)])
                if REM:
                    @pl.when(s == NS - 1)
                    def _():
                        pltpu.sync_copy(z.at[pl.ds(RPT * NS, REM)],
                                        acc.at[pl.ds(RPT * NS, REM)])
        pltpu.sync_copy(x0, xt0)
        pltpu.sync_copy(x1, xt1)
        pltpu.sync_copy(x2, xt2)
        pltpu.sync_copy(zb.at[pl.ds(0, CH)], cbuf)
        plsc.subcore_barrier()

        base = wid * EP
        xts = (xt0, xt1, xt2)
        rv = (rowv0, rowv1)
        iv = (idxv0, idxv1)
        cw = (cwv0, cwv1)
        sL = (sL0, sL1)
        sX = (sX0, sX1)
        NG = NCHK // 2

        for p in (0, 1):
            off = base + p * CH
            pltpu.async_copy(rowi.at[pl.ds(off, CH)], rv[p], sL[p])
            pltpu.async_copy(cw1d.at[pl.ds(off, CH)], cw[p], sL[p])

        def body(g, carry):
            for p in (0, 1):
                off = base + (g * 2 + p) * CH
                pltpu.make_async_copy(
                    rowi.at[pl.ds(off, CH)], rv[p], sL[p]).wait()
                pltpu.make_async_copy(
                    cw1d.at[pl.ds(off, CH)], cw[p], sL[p]).wait()

                if p == 0:
                    @pl.when(g > 0)
                    def _():
                        pltpu.make_async_copy(cbuf, acc.at[iv[p]], sS).wait()
                else:
                    pltpu.make_async_copy(cbuf, acc.at[iv[p]], sS).wait()

                pltpu.async_copy(coli.at[pl.ds(off, CH)], iv[p], sX[p])
                for k in range(CH // 16):
                    r16 = rv[p][pl.ds(k * 16, 16)]
                    c16 = cw[p][pl.ds(k * 16, 16)]
                    e16 = lax.broadcasted_iota(jnp.int32, (16,), 0) + (k * 16)
                    for kk in range(3):
                        v16 = plsc.load_gather(xts[kk], [r16])
                        plsc.store_scatter(
                            cbuf, [e16, jnp.full((16,), kk, jnp.int32)],
                            v16 * c16)
                    plsc.store_scatter(
                        cbuf, [e16, jnp.full((16,), 3, jnp.int32)], c16)
                pltpu.make_async_copy(
                    coli.at[pl.ds(off, CH)], iv[p], sX[p]).wait()
                pltpu.async_copy(cbuf, acc.at[iv[p]], sS, add=True)

                @pl.when(g < NG - 1)
                def _():
                    offn = off + 2 * CH
                    pltpu.async_copy(rowi.at[pl.ds(offn, CH)], rv[p], sL[p])
                    pltpu.async_copy(cw1d.at[pl.ds(offn, CH)], cw[p], sL[p])
            return carry

        lax.fori_loop(0, NG, body, 0)
        pltpu.make_async_copy(cbuf, acc.at[iv[1]], sS).wait()
        plsc.subcore_barrier()

        @pl.when(c == 0)
        def _():
            pltpu.sync_copy(acc.at[pl.ds(r0, RPT)], p0.at[pl.ds(r0, RPT)])
            if REM:
                @pl.when(s == NS - 1)
                def _():
                    pltpu.sync_copy(acc.at[pl.ds(RPT * NS, REM)],
                                    p0.at[pl.ds(RPT * NS, REM)])

        @pl.when(c == 1)
        def _():
            pltpu.sync_copy(acc.at[pl.ds(r0, RPT)], p1.at[pl.ds(r0, RPT)])
            if REM:
                @pl.when(s == NS - 1)
                def _():
                    pltpu.sync_copy(acc.at[pl.ds(RPT * NS, REM)],
                                    p1.at[pl.ds(RPT * NS, REM)])

    return coord_k


def _k4_body(h_ref, pm0_ref, pm1_ref, pc0_ref, pc1_ref, x_ref,
             wn1a_ref, wn1b_ref, bn1_ref, wn2_ref, bn2_ref, g_ref, b_ref,
             hout_ref, xout_ref):
    hb = h_ref[...]
    agg = pm0_ref[...] + pm1_ref[...]
    t = _dot(_silu(_dot(hb, wn1a_ref[...]) + _dot(agg, wn1b_ref[...])
                   + bn1_ref[...]), wn2_ref[...]) + bn2_ref[...]
    pre = hb + t
    mean = jnp.mean(pre, axis=-1, keepdims=True)
    cen = pre - mean
    var = jnp.mean(cen * cen, axis=-1, keepdims=True)
    hout_ref[...] = cen * lax.rsqrt(var + 1e-5) * g_ref[...] + b_ref[...]
    aggc = pc0_ref[...] + pc1_ref[...]
    xb = x_ref[...]
    xout_ref[...] = xb + aggc[:, :3] - xb * aggc[:, 3:4]


def kernel(h, x, edge_index, edge_dist, W_e1, b_e1, W_e2, b_e2, W_a, b_a,
           W_n1, b_n1, W_n2, b_n2, W_c1, b_c1, W_c2, gamma, beta):
    N, D = h.shape
    E = edge_dist.shape[0]
    E_pad = -(-E // (NW * CH * 2)) * (NW * CH * 2)
    pad = E_pad - E

    ei = edge_index.astype(jnp.int32)
    spread = jnp.arange(pad, dtype=jnp.int32) % N
    rowp = jnp.concatenate([ei[0], spread])
    colp = jnp.concatenate([ei[1], spread])
    distp = jnp.concatenate([edge_dist, jnp.zeros((pad,), F32)])

    w1a = W_e1[:D]
    w1b = W_e1[D:2 * D]
    w1c = W_e1[2 * D:2 * D + 1]
    b1 = b_e1[None, :]

    nb = N // NBLK
    hr, hc = pl.pallas_call(
        _k0_body,
        grid=(nb,),
        in_specs=[pl.BlockSpec((NBLK, D), lambda i: (i, 0)),
                  pl.BlockSpec((D, D), lambda i: (0, 0)),
                  pl.BlockSpec((D, D), lambda i: (0, 0)),
                  pl.BlockSpec((1, D), lambda i: (0, 0))],
        out_specs=[pl.BlockSpec((NBLK, D), lambda i: (i, 0)),
                   pl.BlockSpec((NBLK, D), lambda i: (i, 0))],
        out_shape=[jax.ShapeDtypeStruct((N, D), F32),
                   jax.ShapeDtypeStruct((N, D), F32)],
    )(h, w1a, w1b, b1)

    Q = 4
    H = E_pad // Q
    eb = H // BLK
    zm = jnp.zeros((N, D), F32)
    gather_f = _make_gather(N, D, H)
    scat_f = _make_scatter(N, D, H)
    coord_f = _make_coord_scatter(N, D, H)
    pms, pcs = (zm, zm), (zm, zm)
    for hv in range(Q):
        rowh = rowp[hv * H:(hv + 1) * H]
        colh = colp[hv * H:(hv + 1) * H]
        disth = distp[hv * H:(hv + 1) * H]
        t1 = gather_f(hr, hc, rowh, colh)
        msg, cwo = pl.pallas_call(
            _make_edge(E, D, hv * H),
            grid=(eb,),
            in_specs=[pl.BlockSpec((BLK, D), lambda i: (i, 0)),
                      pl.BlockSpec((BLK, 1), lambda i: (i, 0)),
                      pl.BlockSpec((1, D), lambda i: (0, 0)),
                      pl.BlockSpec((D, D), lambda i: (0, 0)),
                      pl.BlockSpec((1, D), lambda i: (0, 0)),
                      pl.BlockSpec((D, 1), lambda i: (0, 0)),
                      pl.BlockSpec((1, 1), lambda i: (0, 0)),
                      pl.BlockSpec((D, D), lambda i: (0, 0)),
                      pl.BlockSpec((1, D), lambda i: (0, 0)),
                      pl.BlockSpec((D, 1), lambda i: (0, 0))],
            out_specs=[pl.BlockSpec((BLK, D), lambda i: (i, 0)),
                       pl.BlockSpec((1, 1, BLK), lambda i: (i, 0, 0))],
            out_shape=[jax.ShapeDtypeStruct((H, D), F32),
                       jax.ShapeDtypeStruct((eb, 1, BLK), F32)],
        )(t1, disth[:, None], w1c, W_e2, b_e2[None, :], W_a, b_a[None, :],
          W_c1, b_c1[None, :], W_c2)
        pms = scat_f(msg, colh, pms[0], pms[1])
        pcs = coord_f(cwo.reshape(H), rowh, colh,
                      x[:, 0], x[:, 1], x[:, 2], pcs[0], pcs[1], zm)
    pm0, pm1 = pms
    pc0, pc1 = pcs

    h_out, x_new = pl.pallas_call(
        _k4_body,
        grid=(nb,),
        in_specs=[pl.BlockSpec((NBLK, D), lambda i: (i, 0)),
                  pl.BlockSpec((NBLK, D), lambda i: (i, 0)),
                  pl.BlockSpec((NBLK, D), lambda i: (i, 0)),
                  pl.BlockSpec((NBLK, 128), lambda i: (i, 0)),
                  pl.BlockSpec((NBLK, 128), lambda i: (i, 0)),
                  pl.BlockSpec((NBLK, 3), lambda i: (i, 0)),
                  pl.BlockSpec((D, D), lambda i: (0, 0)),
                  pl.BlockSpec((D, D), lambda i: (0, 0)),
                  pl.BlockSpec((1, D), lambda i: (0, 0)),
                  pl.BlockSpec((D, D), lambda i: (0, 0)),
                  pl.BlockSpec((1, D), lambda i: (0, 0)),
                  pl.BlockSpec((1, D), lambda i: (0, 0)),
                  pl.BlockSpec((1, D), lambda i: (0, 0))],
        out_specs=[pl.BlockSpec((NBLK, D), lambda i: (i, 0)),
                   pl.BlockSpec((NBLK, 3), lambda i: (i, 0))],
        out_shape=[jax.ShapeDtypeStruct((N, D), F32),
                   jax.ShapeDtypeStruct((N, 3), F32)],
    )(h, pm0, pm1, pc0, pc1, x,
      W_n1[:D], W_n1[D:], b_n1[None, :],
      W_n2, b_n2[None, :], gamma[None, :], beta[None, :])

    return (h_out, x_new)
